# baseline (speedup 1.0000x reference)
import functools

import jax
import jax.numpy as jnp
import numpy as np
from jax import lax
from jax.experimental import pallas as pl
from jax.experimental.pallas import tpu as pltpu
from jax.experimental.pallas import tpu_sc as plsc

NS = 16
NV = 4
N_NODES = 10000
N_EDGES = 160000
SH_DIM = 9
EDGE_FEAT = 3 * NS
MSG_W = 128
OUT_W = NS + 3 * NV + 5 * NV
INV = 1.0 / np.sqrt(NS)

_SC_PARAMS = pltpu.CompilerParams(use_tc_tiling_on_sc=False)

NUM_WORKERS = 32
PER_W = N_EDGES // NUM_WORKERS
CHUNK = 128
NCH = PER_W // CHUNK
TAIL = PER_W - NCH * CHUNK

_E1 = np.zeros((NS, NS * NS), np.float32)
_E2 = np.zeros((128, NS * NS), np.float32)
for _k in range(NS):
    _E1[_k, _k * NS:(_k + 1) * NS] = 1.0
    for _i in range(NS):
        _E2[_i, _k * NS + _i] = 1.0

_SEL = np.zeros((3, 128, EDGE_FEAT), np.float32)
_EMASK = np.zeros((3, 128, 128), np.float32)
for _r in range(3):
    for _l in range(128):
        _g = 128 * _r + _l
        _e, _f = divmod(_g, EDGE_FEAT)
        _SEL[_r, _l, _f] = 1.0
        _EMASK[_r, _l, _e * NS:(_e + 1) * NS] = 1.0
_G16 = np.zeros((NS, 128), np.float32)
for _o in range(NS):
    _G16[_o, _o::NS] = 1.0
_C16 = np.zeros((128, NS), np.float32)
for _e in range(8):
    for _k in range(NS):
        _C16[_e * NS + _k, _k] = 1.0


def _assemble_mlp1(fc_w1, fc_b1):
    d = jnp.einsum('rlf,fo->rlo', jnp.asarray(_SEL), fc_w1)
    m = jnp.einsum('rlo,om->rlm', d, jnp.asarray(_G16)) * jnp.asarray(_EMASK)
    bias_t = jnp.tile(fc_b1, 8).reshape(1, 128)
    return m, bias_t

_ESH = np.zeros((SH_DIM, MSG_W), np.float32)
_ESH[0, 0:NS] = 1.0
for _v in range(NV):
    for _j in range(3):
        _ESH[1 + _j, NS + _v * 3 + _j] = 1.0
    for _j in range(5):
        _ESH[4 + _j, NS + 3 * NV + _v * 5 + _j] = 1.0

def _assemble_weights(fc_w2, fc_b2):
    w0 = fc_w2[:, :NS * NS].reshape(NS * NS, NS)
    w1 = jnp.repeat(fc_w2[:, NS * NS:NS * NS + NS * NV].reshape(NS * NS, NV),
                    3, axis=1)
    w2 = jnp.repeat(fc_w2[:, NS * NS + NS * NV:].reshape(NS * NS, NV),
                    5, axis=1)
    wc = jnp.concatenate([w0, w1, w2], axis=1) * INV
    b0 = fc_b2[:NS * NS].reshape(NS, NS)
    b1 = jnp.repeat(fc_b2[NS * NS:NS * NS + NS * NV].reshape(NS, NV), 3,
                    axis=1)
    b2 = jnp.repeat(fc_b2[NS * NS + NS * NV:].reshape(NS, NV), 5, axis=1)
    wb = jnp.concatenate([b0, b1, b2], axis=1) * INV
    wc = jnp.pad(wc, ((0, 0), (0, MSG_W - OUT_W)))
    wb = jnp.pad(wb, ((0, 112), (0, MSG_W - OUT_W)))
    return wc, wb


def _sc_gather(node_pad, src):
    mesh = plsc.VectorSubcoreMesh(core_axis_name="c", subcore_axis_name="s")

    @functools.partial(
        pl.kernel,
        out_type=jax.ShapeDtypeStruct((N_EDGES, 128), jnp.float32),
        mesh=mesh,
        scratch_types=[
            pltpu.VMEM((CHUNK,), jnp.int32),
            pltpu.VMEM((CHUNK, 128), jnp.float32),
            pltpu.VMEM((CHUNK,), jnp.int32),
            pltpu.VMEM((CHUNK, 128), jnp.float32),
            pltpu.VMEM((TAIL,), jnp.int32),
            pltpu.VMEM((TAIL, 128), jnp.float32),
            pltpu.SemaphoreType.DMA,
            pltpu.SemaphoreType.DMA,
            pltpu.SemaphoreType.DMA,
            pltpu.SemaphoreType.DMA,
        ],
        compiler_params=_SC_PARAMS,
    )
    def k(node_hbm, src_hbm, out_hbm, idx0, rows0, idx1, rows1, idxt, rowst,
          sg0, sg1, sw0, sw1):
        cid = lax.axis_index("c")
        sid = lax.axis_index("s")
        base = (sid * 2 + cid) * PER_W

        def fire(c, idx_v, rows_v, sg):
            off = base + c * CHUNK
            pltpu.sync_copy(src_hbm.at[pl.ds(off, CHUNK)], idx_v)
            pltpu.async_copy(node_hbm.at[idx_v], rows_v, sg)

        def flush(c, idx_v, rows_v, sg, sw):
            pltpu.make_async_copy(node_hbm.at[idx_v], rows_v, sg).wait()
            off = base + c * CHUNK
            pltpu.async_copy(rows_v, out_hbm.at[pl.ds(off, CHUNK)], sw)

        def wait_wb(rows_v, sw):
            pltpu.make_async_copy(rows_v, out_hbm.at[pl.ds(base, CHUNK)],
                                  sw).wait()

        fire(0, idx0, rows0, sg0)
        fire(1, idx1, rows1, sg1)

        @pl.loop(0, NCH - 3, step=2)
        def _(c):
            flush(c, idx0, rows0, sg0, sw0)
            flush(c + 1, idx1, rows1, sg1, sw1)
            wait_wb(rows0, sw0)
            fire(c + 2, idx0, rows0, sg0)
            wait_wb(rows1, sw1)
            fire(c + 3, idx1, rows1, sg1)

        flush(NCH - 3, idx0, rows0, sg0, sw0)
        flush(NCH - 2, idx1, rows1, sg1, sw1)
        wait_wb(rows0, sw0)
        fire(NCH - 1, idx0, rows0, sg0)
        flush(NCH - 1, idx0, rows0, sg0, sw0)
        off = base + NCH * CHUNK
        pltpu.sync_copy(src_hbm.at[pl.ds(off, TAIL)], idxt)
        pltpu.async_copy(node_hbm.at[idxt], rowst, sg0).wait()
        pltpu.sync_copy(rowst, out_hbm.at[pl.ds(off, TAIL)])
        wait_wb(rows1, sw1)
        wait_wb(rows0, sw0)

    return k(node_pad, src)


_B_EDGE = 3200


def _tc_msg_body(x_ref, ea_ref, sh_ref, m_ref, bt_ref, c16_ref, e1_ref,
                 e2_ref, wc_ref, wb_ref, esh_ref, out_ref):
    f32 = jnp.float32
    dot = functools.partial(jnp.dot, preferred_element_type=f32)
    hp = jnp.maximum(
        dot(ea_ref[:, 0:128], m_ref[0]) + dot(ea_ref[:, 128:256], m_ref[1])
        + dot(ea_ref[:, 256:384], m_ref[2]) + bt_ref[...], 0.0)
    b8 = _B_EDGE // 8
    hm = jnp.broadcast_to(hp[:, None, :], (b8, 8, 128)).reshape(_B_EDGE, 128)
    row8 = lax.broadcasted_iota(jnp.int32, (_B_EDGE, 128), 0) % 8
    lane16 = lax.broadcasted_iota(jnp.int32, (_B_EDGE, 128), 1) // NS
    h16 = dot(jnp.where(lane16 == row8, hm, 0.0), c16_ref[...])
    he = dot(h16, e1_ref[...])
    xe = dot(x_ref[...], e2_ref[...])
    pre = dot(xe * he, wc_ref[...]) + dot(x_ref[...], wb_ref[...])
    s = dot(sh_ref[...], esh_ref[...])
    lane = lax.broadcasted_iota(jnp.int32, out_ref.shape, 1)
    out_ref[...] = pre * s + jnp.where(lane == OUT_W, 1.0, 0.0)


def _tc_msg(x128, ea_r, edge_sh, m, bias_t, wc, wb):
    grid = (N_EDGES // _B_EDGE,)
    full = lambda i: (0, 0)
    full3 = lambda i: (0, 0, 0)
    b8 = _B_EDGE // 8
    return pl.pallas_call(
        _tc_msg_body,
        grid=grid,
        in_specs=[
            pl.BlockSpec((_B_EDGE, 128), lambda i: (i, 0)),
            pl.BlockSpec((b8, 3 * 128), lambda i: (i, 0)),
            pl.BlockSpec((_B_EDGE, SH_DIM), lambda i: (i, 0)),
            pl.BlockSpec((3, 128, 128), full3),
            pl.BlockSpec((1, 128), full),
            pl.BlockSpec((128, NS), full),
            pl.BlockSpec((NS, NS * NS), full),
            pl.BlockSpec((128, NS * NS), full),
            pl.BlockSpec((NS * NS, MSG_W), full),
            pl.BlockSpec((128, MSG_W), full),
            pl.BlockSpec((SH_DIM, MSG_W), full),
        ],
        out_specs=pl.BlockSpec((_B_EDGE, MSG_W), lambda i: (i, 0)),
        out_shape=jax.ShapeDtypeStruct((N_EDGES, MSG_W), jnp.float32),
    )(x128, ea_r, edge_sh, m, bias_t, jnp.asarray(_C16), jnp.asarray(_E1),
      jnp.asarray(_E2), wc, wb, jnp.asarray(_ESH))


def _sc_scatter(msg, dst, zer):
    mesh = plsc.VectorSubcoreMesh(core_axis_name="c", subcore_axis_name="s")
    half = N_EDGES // 2
    per_s = half // 16

    @functools.partial(
        pl.kernel,
        out_type=jax.ShapeDtypeStruct((2, N_NODES, MSG_W), jnp.float32),
        mesh=mesh,
        scratch_types=[
            pltpu.VMEM_SHARED((N_NODES, MSG_W), jnp.float32),
            pltpu.VMEM((CHUNK,), jnp.int32),
            pltpu.VMEM((CHUNK, MSG_W), jnp.float32),
            pltpu.VMEM((CHUNK,), jnp.int32),
            pltpu.VMEM((CHUNK, MSG_W), jnp.float32),
            pltpu.VMEM((TAIL,), jnp.int32),
            pltpu.VMEM((TAIL, MSG_W), jnp.float32),
            pltpu.SemaphoreType.DMA,
            pltpu.SemaphoreType.DMA,
            pltpu.SemaphoreType.DMA,
            pltpu.SemaphoreType.DMA,
        ],
        compiler_params=_SC_PARAMS,
    )
    def k(msg_hbm, dst_hbm, zer_hbm, out_hbm, acc, idx0, msg0, idx1, msg1,
          idxt_v, msgt_v, sl0, sl1, ss0, ss1):
        cid = lax.axis_index("c")
        sid = lax.axis_index("s")

        @pl.when(sid == 0)
        def _():
            pltpu.sync_copy(zer_hbm, acc)

        plsc.subcore_barrier()
        base = cid * half + sid * per_s

        def fire_load(c, idx_v, msg_v, sem):
            off = base + c * CHUNK
            pltpu.async_copy(dst_hbm.at[pl.ds(off, CHUNK)], idx_v, sem)
            pltpu.async_copy(msg_hbm.at[pl.ds(off, CHUNK)], msg_v, sem)

        def wait_load(idx_v, msg_v, sem):
            pltpu.make_async_copy(dst_hbm.at[pl.ds(base, CHUNK)], idx_v,
                                  sem).wait()
            pltpu.make_async_copy(msg_hbm.at[pl.ds(base, CHUNK)], msg_v,
                                  sem).wait()

        def fire_scatter(idx_v, msg_v, sem):
            pltpu.async_copy(msg_v, acc.at[idx_v], sem, add=True)

        def wait_scatter(idx_v, msg_v, sem):
            pltpu.make_async_copy(msg_v, acc.at[idx_v], sem).wait()

        fire_load(0, idx0, msg0, sl0)
        fire_load(1, idx1, msg1, sl1)

        @pl.loop(0, NCH - 3, step=2)
        def _(c):
            wait_load(idx0, msg0, sl0)
            fire_scatter(idx0, msg0, ss0)
            wait_load(idx1, msg1, sl1)
            fire_scatter(idx1, msg1, ss1)
            wait_scatter(idx0, msg0, ss0)
            fire_load(c + 2, idx0, msg0, sl0)
            wait_scatter(idx1, msg1, ss1)
            fire_load(c + 3, idx1, msg1, sl1)

        wait_load(idx0, msg0, sl0)
        fire_scatter(idx0, msg0, ss0)
        wait_load(idx1, msg1, sl1)
        fire_scatter(idx1, msg1, ss1)
        wait_scatter(idx0, msg0, ss0)
        fire_load(NCH - 1, idx0, msg0, sl0)
        wait_load(idx0, msg0, sl0)
        fire_scatter(idx0, msg0, ss0)
        off = base + NCH * CHUNK
        pltpu.sync_copy(dst_hbm.at[pl.ds(off, TAIL)], idxt_v)
        pltpu.sync_copy(msg_hbm.at[pl.ds(off, TAIL)], msgt_v)
        pltpu.sync_copy(msgt_v, acc.at[idxt_v], add=True)
        wait_scatter(idx1, msg1, ss1)
        wait_scatter(idx0, msg0, ss0)

        plsc.subcore_barrier()

        @pl.when(sid == 0)
        def _():
            pltpu.sync_copy(acc, out_hbm.at[cid])

    return k(msg, dst, zer)


_B_NODE = 2000


def _tc_fin_body(p_ref, out_ref):
    s = p_ref[0] + p_ref[1]
    cnt = jnp.clip(s[:, OUT_W:OUT_W + 1], 1.0, None)
    out_ref[...] = s[:, 0:OUT_W] / cnt


def _tc_finalize(part):
    return pl.pallas_call(
        _tc_fin_body,
        grid=(N_NODES // _B_NODE,),
        in_specs=[pl.BlockSpec((2, _B_NODE, MSG_W), lambda i: (0, i, 0))],
        out_specs=pl.BlockSpec((_B_NODE, OUT_W), lambda i: (i, 0)),
        out_shape=jax.ShapeDtypeStruct((N_NODES, OUT_W), jnp.float32),
    )(part)


def kernel(node_attr, edge_index, edge_attr, edge_sh, fc_w1, fc_b1, fc_w2,
           fc_b2):
    src = edge_index[0]
    dst = edge_index[1]
    node_pad = jnp.pad(node_attr, ((0, 0), (0, 128 - NS)))
    x128 = _sc_gather(node_pad, src)
    ea_r = edge_attr.reshape(N_EDGES // 8, 8 * EDGE_FEAT)
    wc, wb = _assemble_weights(fc_w2, fc_b2)
    m, bias_t = _assemble_mlp1(fc_w1, fc_b1)
    msg = _tc_msg(x128, ea_r, edge_sh, m, bias_t, wc, wb)
    zer = jnp.zeros((N_NODES, MSG_W), jnp.float32)
    part = _sc_scatter(msg, dst, zer)
    return _tc_finalize(part)


# speedup vs baseline: 1.4386x; 1.4386x over previous
import functools

import jax
import jax.numpy as jnp
import numpy as np
from jax import lax
from jax.experimental import pallas as pl
from jax.experimental.pallas import tpu as pltpu
from jax.experimental.pallas import tpu_sc as plsc

NS = 16
NV = 4
N_NODES = 10000
N_EDGES = 160000
SH_DIM = 9
EDGE_FEAT = 3 * NS
MSG_W = 128
OUT_W = NS + 3 * NV + 5 * NV
INV = 1.0 / np.sqrt(NS)

_SC_PARAMS = pltpu.CompilerParams(use_tc_tiling_on_sc=False)

NUM_WORKERS = 32
PER_W = N_EDGES // NUM_WORKERS
CHUNK = 128
NCH = PER_W // CHUNK
TAIL = PER_W - NCH * CHUNK

_E1 = np.zeros((NS, NS * NS), np.float32)
_E2 = np.zeros((128, NS * NS), np.float32)
for _k in range(NS):
    _E1[_k, _k * NS:(_k + 1) * NS] = 1.0
    for _i in range(NS):
        _E2[_i, _k * NS + _i] = 1.0


_ESH = np.zeros((SH_DIM, MSG_W), np.float32)
_ESH[0, 0:NS] = 1.0
for _v in range(NV):
    for _j in range(3):
        _ESH[1 + _j, NS + _v * 3 + _j] = 1.0
    for _j in range(5):
        _ESH[4 + _j, NS + 3 * NV + _v * 5 + _j] = 1.0

def _assemble_weights(fc_w2, fc_b2):
    w0 = fc_w2[:, :NS * NS].reshape(NS * NS, NS)
    w1 = jnp.repeat(fc_w2[:, NS * NS:NS * NS + NS * NV].reshape(NS * NS, NV),
                    3, axis=1)
    w2 = jnp.repeat(fc_w2[:, NS * NS + NS * NV:].reshape(NS * NS, NV),
                    5, axis=1)
    wc = jnp.concatenate([w0, w1, w2], axis=1) * INV
    b0 = fc_b2[:NS * NS].reshape(NS, NS)
    b1 = jnp.repeat(fc_b2[NS * NS:NS * NS + NS * NV].reshape(NS, NV), 3,
                    axis=1)
    b2 = jnp.repeat(fc_b2[NS * NS + NS * NV:].reshape(NS, NV), 5, axis=1)
    wb = jnp.concatenate([b0, b1, b2], axis=1) * INV
    wc = jnp.pad(wc, ((0, 0), (0, MSG_W - OUT_W)))
    wb = jnp.pad(wb, ((0, 112), (0, MSG_W - OUT_W)))
    return wc, wb


def _sc_gather(node_pad, src):
    mesh = plsc.VectorSubcoreMesh(core_axis_name="c", subcore_axis_name="s")

    @functools.partial(
        pl.kernel,
        out_type=jax.ShapeDtypeStruct((N_EDGES, 128), jnp.float32),
        mesh=mesh,
        scratch_types=[
            pltpu.VMEM((CHUNK,), jnp.int32),
            pltpu.VMEM((CHUNK, 128), jnp.float32),
            pltpu.VMEM((CHUNK,), jnp.int32),
            pltpu.VMEM((CHUNK, 128), jnp.float32),
            pltpu.VMEM((TAIL,), jnp.int32),
            pltpu.VMEM((TAIL, 128), jnp.float32),
            pltpu.SemaphoreType.DMA,
            pltpu.SemaphoreType.DMA,
            pltpu.SemaphoreType.DMA,
            pltpu.SemaphoreType.DMA,
        ],
        compiler_params=_SC_PARAMS,
    )
    def k(node_hbm, src_hbm, out_hbm, idx0, rows0, idx1, rows1, idxt, rowst,
          sg0, sg1, sw0, sw1):
        cid = lax.axis_index("c")
        sid = lax.axis_index("s")
        base = (sid * 2 + cid) * PER_W

        def fire(c, idx_v, rows_v, sg):
            off = base + c * CHUNK
            pltpu.sync_copy(src_hbm.at[pl.ds(off, CHUNK)], idx_v)
            pltpu.async_copy(node_hbm.at[idx_v], rows_v, sg)

        def flush(c, idx_v, rows_v, sg, sw):
            pltpu.make_async_copy(node_hbm.at[idx_v], rows_v, sg).wait()
            off = base + c * CHUNK
            pltpu.async_copy(rows_v, out_hbm.at[pl.ds(off, CHUNK)], sw)

        def wait_wb(rows_v, sw):
            pltpu.make_async_copy(rows_v, out_hbm.at[pl.ds(base, CHUNK)],
                                  sw).wait()

        fire(0, idx0, rows0, sg0)
        fire(1, idx1, rows1, sg1)

        @pl.loop(0, NCH - 3, step=2)
        def _(c):
            flush(c, idx0, rows0, sg0, sw0)
            flush(c + 1, idx1, rows1, sg1, sw1)
            wait_wb(rows0, sw0)
            fire(c + 2, idx0, rows0, sg0)
            wait_wb(rows1, sw1)
            fire(c + 3, idx1, rows1, sg1)

        flush(NCH - 3, idx0, rows0, sg0, sw0)
        flush(NCH - 2, idx1, rows1, sg1, sw1)
        wait_wb(rows0, sw0)
        fire(NCH - 1, idx0, rows0, sg0)
        flush(NCH - 1, idx0, rows0, sg0, sw0)
        off = base + NCH * CHUNK
        pltpu.sync_copy(src_hbm.at[pl.ds(off, TAIL)], idxt)
        pltpu.async_copy(node_hbm.at[idxt], rowst, sg0).wait()
        pltpu.sync_copy(rowst, out_hbm.at[pl.ds(off, TAIL)])
        wait_wb(rows1, sw1)
        wait_wb(rows0, sw0)

    return k(node_pad, src)


_B_EDGE = 6400


def _tc_msg_body(x_ref, eat_ref, sht_ref, w1t_ref, b1_ref, e1_ref, e2_ref,
                 wc_ref, wb_ref, esh_ref, out_ref):
    f32 = jnp.float32
    dot = functools.partial(jnp.dot, preferred_element_type=f32)
    ht = jnp.maximum(dot(w1t_ref[...], eat_ref[...]) + b1_ref[...], 0.0)
    h = ht.T
    sh = sht_ref[...].T
    he = dot(h, e1_ref[...])
    xe = dot(x_ref[...], e2_ref[...])
    pre = dot(xe * he, wc_ref[...]) + dot(x_ref[...], wb_ref[...])
    s = dot(sh, esh_ref[...])
    lane = lax.broadcasted_iota(jnp.int32, out_ref.shape, 1)
    out_ref[...] = pre * s + jnp.where(lane == OUT_W, 1.0, 0.0)


def _tc_msg(x128, eat, sht, w1t, fc_b1, wc, wb):
    grid = (N_EDGES // _B_EDGE,)
    full = lambda i: (0, 0)
    return pl.pallas_call(
        _tc_msg_body,
        grid=grid,
        in_specs=[
            pl.BlockSpec((_B_EDGE, 128), lambda i: (i, 0)),
            pl.BlockSpec((EDGE_FEAT, _B_EDGE), lambda i: (0, i)),
            pl.BlockSpec((SH_DIM, _B_EDGE), lambda i: (0, i)),
            pl.BlockSpec((NS, EDGE_FEAT), full),
            pl.BlockSpec((NS, 1), full),
            pl.BlockSpec((NS, NS * NS), full),
            pl.BlockSpec((128, NS * NS), full),
            pl.BlockSpec((NS * NS, MSG_W), full),
            pl.BlockSpec((128, MSG_W), full),
            pl.BlockSpec((SH_DIM, MSG_W), full),
        ],
        out_specs=pl.BlockSpec((_B_EDGE, MSG_W), lambda i: (i, 0)),
        out_shape=jax.ShapeDtypeStruct((N_EDGES, MSG_W), jnp.float32),
    )(x128, eat, sht, w1t, fc_b1.reshape(NS, 1), jnp.asarray(_E1),
      jnp.asarray(_E2), wc, wb, jnp.asarray(_ESH))


def _sc_scatter(msg, dst, zer):
    mesh = plsc.VectorSubcoreMesh(core_axis_name="c", subcore_axis_name="s")
    half = N_EDGES // 2
    per_s = half // 16

    @functools.partial(
        pl.kernel,
        out_type=jax.ShapeDtypeStruct((2, N_NODES, MSG_W), jnp.float32),
        mesh=mesh,
        scratch_types=[
            pltpu.VMEM_SHARED((N_NODES, MSG_W), jnp.float32),
            pltpu.VMEM((CHUNK,), jnp.int32),
            pltpu.VMEM((CHUNK, MSG_W), jnp.float32),
            pltpu.VMEM((CHUNK,), jnp.int32),
            pltpu.VMEM((CHUNK, MSG_W), jnp.float32),
            pltpu.VMEM((TAIL,), jnp.int32),
            pltpu.VMEM((TAIL, MSG_W), jnp.float32),
            pltpu.SemaphoreType.DMA,
            pltpu.SemaphoreType.DMA,
            pltpu.SemaphoreType.DMA,
            pltpu.SemaphoreType.DMA,
        ],
        compiler_params=_SC_PARAMS,
    )
    def k(msg_hbm, dst_hbm, zer_hbm, out_hbm, acc, idx0, msg0, idx1, msg1,
          idxt_v, msgt_v, sl0, sl1, ss0, ss1):
        cid = lax.axis_index("c")
        sid = lax.axis_index("s")

        @pl.when(sid == 0)
        def _():
            pltpu.sync_copy(zer_hbm, acc)

        plsc.subcore_barrier()
        base = cid * half + sid * per_s

        def fire_load(c, idx_v, msg_v, sem):
            off = base + c * CHUNK
            pltpu.async_copy(dst_hbm.at[pl.ds(off, CHUNK)], idx_v, sem)
            pltpu.async_copy(msg_hbm.at[pl.ds(off, CHUNK)], msg_v, sem)

        def wait_load(idx_v, msg_v, sem):
            pltpu.make_async_copy(dst_hbm.at[pl.ds(base, CHUNK)], idx_v,
                                  sem).wait()
            pltpu.make_async_copy(msg_hbm.at[pl.ds(base, CHUNK)], msg_v,
                                  sem).wait()

        def fire_scatter(idx_v, msg_v, sem):
            pltpu.async_copy(msg_v, acc.at[idx_v], sem, add=True)

        def wait_scatter(idx_v, msg_v, sem):
            pltpu.make_async_copy(msg_v, acc.at[idx_v], sem).wait()

        fire_load(0, idx0, msg0, sl0)
        fire_load(1, idx1, msg1, sl1)

        @pl.loop(0, NCH - 3, step=2)
        def _(c):
            wait_load(idx0, msg0, sl0)
            fire_scatter(idx0, msg0, ss0)
            wait_load(idx1, msg1, sl1)
            fire_scatter(idx1, msg1, ss1)
            wait_scatter(idx0, msg0, ss0)
            fire_load(c + 2, idx0, msg0, sl0)
            wait_scatter(idx1, msg1, ss1)
            fire_load(c + 3, idx1, msg1, sl1)

        wait_load(idx0, msg0, sl0)
        fire_scatter(idx0, msg0, ss0)
        wait_load(idx1, msg1, sl1)
        fire_scatter(idx1, msg1, ss1)
        wait_scatter(idx0, msg0, ss0)
        fire_load(NCH - 1, idx0, msg0, sl0)
        wait_load(idx0, msg0, sl0)
        fire_scatter(idx0, msg0, ss0)
        off = base + NCH * CHUNK
        pltpu.sync_copy(dst_hbm.at[pl.ds(off, TAIL)], idxt_v)
        pltpu.sync_copy(msg_hbm.at[pl.ds(off, TAIL)], msgt_v)
        pltpu.sync_copy(msgt_v, acc.at[idxt_v], add=True)
        wait_scatter(idx1, msg1, ss1)
        wait_scatter(idx0, msg0, ss0)

        plsc.subcore_barrier()

        @pl.when(sid == 0)
        def _():
            pltpu.sync_copy(acc, out_hbm.at[cid])

    return k(msg, dst, zer)


_B_NODE = 2000


def _tc_fin_body(p_ref, out_ref):
    s = p_ref[0] + p_ref[1]
    cnt = jnp.clip(s[:, OUT_W:OUT_W + 1], 1.0, None)
    out_ref[...] = s[:, 0:OUT_W] / cnt


def _tc_finalize(part):
    return pl.pallas_call(
        _tc_fin_body,
        grid=(N_NODES // _B_NODE,),
        in_specs=[pl.BlockSpec((2, _B_NODE, MSG_W), lambda i: (0, i, 0))],
        out_specs=pl.BlockSpec((_B_NODE, OUT_W), lambda i: (i, 0)),
        out_shape=jax.ShapeDtypeStruct((N_NODES, OUT_W), jnp.float32),
    )(part)


def kernel(node_attr, edge_index, edge_attr, edge_sh, fc_w1, fc_b1, fc_w2,
           fc_b2):
    src = edge_index[0]
    dst = edge_index[1]
    node_pad = jnp.pad(node_attr, ((0, 0), (0, 128 - NS)))
    x128 = _sc_gather(node_pad, src)
    wc, wb = _assemble_weights(fc_w2, fc_b2)
    msg = _tc_msg(x128, edge_attr.T, edge_sh.T, fc_w1.T, fc_b1, wc, wb)
    zer = jnp.zeros((N_NODES, MSG_W), jnp.float32)
    part = _sc_scatter(msg, dst, zer)
    return _tc_finalize(part)


# speedup vs baseline: 1.5778x; 1.0967x over previous
import functools

import jax
import jax.numpy as jnp
import numpy as np
from jax import lax
from jax.experimental import pallas as pl
from jax.experimental.pallas import tpu as pltpu
from jax.experimental.pallas import tpu_sc as plsc

NS = 16
NV = 4
N_NODES = 10000
N_EDGES = 160000
SH_DIM = 9
EDGE_FEAT = 3 * NS
MSG_W = 128
OUT_W = NS + 3 * NV + 5 * NV
INV = 1.0 / np.sqrt(NS)

_SC_PARAMS = pltpu.CompilerParams(use_tc_tiling_on_sc=False)

NUM_WORKERS = 32
PER_W = N_EDGES // NUM_WORKERS
CHUNK = 128
NCH = PER_W // CHUNK
TAIL = PER_W - NCH * CHUNK

_E1 = np.zeros((NS, NS * NS), np.float32)
_E2 = np.zeros((128, NS * NS), np.float32)
for _k in range(NS):
    _E1[_k, _k * NS:(_k + 1) * NS] = 1.0
    for _i in range(NS):
        _E2[_i, _k * NS + _i] = 1.0


_ESH = np.zeros((SH_DIM, MSG_W), np.float32)
_ESH[0, 0:NS] = 1.0
for _v in range(NV):
    for _j in range(3):
        _ESH[1 + _j, NS + _v * 3 + _j] = 1.0
    for _j in range(5):
        _ESH[4 + _j, NS + 3 * NV + _v * 5 + _j] = 1.0

def _assemble_weights(fc_w2, fc_b2):
    w0 = fc_w2[:, :NS * NS].reshape(NS * NS, NS)
    w1 = jnp.repeat(fc_w2[:, NS * NS:NS * NS + NS * NV].reshape(NS * NS, NV),
                    3, axis=1)
    w2 = jnp.repeat(fc_w2[:, NS * NS + NS * NV:].reshape(NS * NS, NV),
                    5, axis=1)
    wc = jnp.concatenate([w0, w1, w2], axis=1) * INV
    b0 = fc_b2[:NS * NS].reshape(NS, NS)
    b1 = jnp.repeat(fc_b2[NS * NS:NS * NS + NS * NV].reshape(NS, NV), 3,
                    axis=1)
    b2 = jnp.repeat(fc_b2[NS * NS + NS * NV:].reshape(NS, NV), 5, axis=1)
    wb = jnp.concatenate([b0, b1, b2], axis=1) * INV
    wc = jnp.pad(wc, ((0, 0), (0, MSG_W - OUT_W)))
    wb = jnp.pad(wb, ((0, 112), (0, MSG_W - OUT_W)))
    return wc, wb


def _sc_gather(node_pad, src, n_e):
    mesh = plsc.VectorSubcoreMesh(core_axis_name="c", subcore_axis_name="s")
    per_w = n_e // NUM_WORKERS
    assert per_w % 8 == 0
    nch = per_w // CHUNK
    assert nch % 2 == 0 and nch >= 2
    tail = per_w - nch * CHUNK

    @functools.partial(
        pl.kernel,
        out_type=jax.ShapeDtypeStruct((n_e, 128), jnp.float32),
        mesh=mesh,
        scratch_types=[
            pltpu.VMEM((CHUNK,), jnp.int32),
            pltpu.VMEM((CHUNK, 128), jnp.float32),
            pltpu.VMEM((CHUNK,), jnp.int32),
            pltpu.VMEM((CHUNK, 128), jnp.float32),
            pltpu.VMEM((tail,), jnp.int32),
            pltpu.VMEM((tail, 128), jnp.float32),
            pltpu.SemaphoreType.DMA,
            pltpu.SemaphoreType.DMA,
            pltpu.SemaphoreType.DMA,
            pltpu.SemaphoreType.DMA,
        ],
        compiler_params=_SC_PARAMS,
    )
    def k(node_hbm, src_hbm, out_hbm, idx0, rows0, idx1, rows1, idxt, rowst,
          sg0, sg1, sw0, sw1):
        cid = lax.axis_index("c")
        sid = lax.axis_index("s")
        base = (sid * 2 + cid) * per_w

        def fire(c, idx_v, rows_v, sg):
            off = base + c * CHUNK
            pltpu.sync_copy(src_hbm.at[pl.ds(off, CHUNK)], idx_v)
            pltpu.async_copy(node_hbm.at[idx_v], rows_v, sg)

        def flush(c, idx_v, rows_v, sg, sw):
            pltpu.make_async_copy(node_hbm.at[idx_v], rows_v, sg).wait()
            off = base + c * CHUNK
            pltpu.async_copy(rows_v, out_hbm.at[pl.ds(off, CHUNK)], sw)

        def wait_wb(rows_v, sw):
            pltpu.make_async_copy(rows_v, out_hbm.at[pl.ds(base, CHUNK)],
                                  sw).wait()

        fire(0, idx0, rows0, sg0)
        fire(1, idx1, rows1, sg1)

        @pl.loop(0, nch - 2, step=2)
        def _(c):
            flush(c, idx0, rows0, sg0, sw0)
            flush(c + 1, idx1, rows1, sg1, sw1)
            wait_wb(rows0, sw0)
            fire(c + 2, idx0, rows0, sg0)
            wait_wb(rows1, sw1)
            fire(c + 3, idx1, rows1, sg1)

        flush(nch - 2, idx0, rows0, sg0, sw0)
        flush(nch - 1, idx1, rows1, sg1, sw1)
        off = base + nch * CHUNK
        pltpu.sync_copy(src_hbm.at[pl.ds(off, tail)], idxt)
        pltpu.async_copy(node_hbm.at[idxt], rowst, sg0).wait()
        pltpu.sync_copy(rowst, out_hbm.at[pl.ds(off, tail)])
        wait_wb(rows1, sw1)
        wait_wb(rows0, sw0)

    return k(node_pad, src)


_B_EDGE = 3200


def _tc_msg_body(x_ref, eat_ref, sht_ref, w1t_ref, b1_ref, e1_ref, e2_ref,
                 wc_ref, wb_ref, esh_ref, out_ref):
    f32 = jnp.float32
    dot = functools.partial(jnp.dot, preferred_element_type=f32)
    ht = jnp.maximum(dot(w1t_ref[...], eat_ref[...]) + b1_ref[...], 0.0)
    h = ht.T
    sh = sht_ref[...].T
    he = dot(h, e1_ref[...])
    xe = dot(x_ref[...], e2_ref[...])
    pre = dot(xe * he, wc_ref[...]) + dot(x_ref[...], wb_ref[...])
    s = dot(sh, esh_ref[...])
    lane = lax.broadcasted_iota(jnp.int32, out_ref.shape, 1)
    out_ref[...] = pre * s + jnp.where(lane == OUT_W, 1.0, 0.0)


def _tc_msg(x128, eat, sht, w1t, fc_b1, wc, wb, n_e, blk_off):
    grid = (n_e // _B_EDGE,)
    full = lambda i: (0, 0)
    return pl.pallas_call(
        _tc_msg_body,
        grid=grid,
        in_specs=[
            pl.BlockSpec((_B_EDGE, 128), lambda i: (i, 0)),
            pl.BlockSpec((EDGE_FEAT, _B_EDGE), lambda i: (0, i + blk_off)),
            pl.BlockSpec((SH_DIM, _B_EDGE), lambda i: (0, i + blk_off)),
            pl.BlockSpec((NS, EDGE_FEAT), full),
            pl.BlockSpec((NS, 1), full),
            pl.BlockSpec((NS, NS * NS), full),
            pl.BlockSpec((128, NS * NS), full),
            pl.BlockSpec((NS * NS, MSG_W), full),
            pl.BlockSpec((128, MSG_W), full),
            pl.BlockSpec((SH_DIM, MSG_W), full),
        ],
        out_specs=pl.BlockSpec((_B_EDGE, MSG_W), lambda i: (i, 0)),
        out_shape=jax.ShapeDtypeStruct((n_e, MSG_W), jnp.float32),
    )(x128, eat, sht, w1t, fc_b1.reshape(NS, 1), jnp.asarray(_E1),
      jnp.asarray(_E2), wc, wb, jnp.asarray(_ESH))


def _sc_scatter(msg, dst, zer, n_e):
    mesh = plsc.VectorSubcoreMesh(core_axis_name="c", subcore_axis_name="s")
    half = n_e // 2
    per_s = half // 16
    assert per_s % 8 == 0
    nch = per_s // CHUNK
    assert nch % 2 == 0 and nch >= 2
    tail = per_s - nch * CHUNK

    @functools.partial(
        pl.kernel,
        out_type=jax.ShapeDtypeStruct((2, N_NODES, MSG_W), jnp.float32),
        mesh=mesh,
        scratch_types=[
            pltpu.VMEM_SHARED((N_NODES, MSG_W), jnp.float32),
            pltpu.VMEM((CHUNK,), jnp.int32),
            pltpu.VMEM((CHUNK, MSG_W), jnp.float32),
            pltpu.VMEM((CHUNK,), jnp.int32),
            pltpu.VMEM((CHUNK, MSG_W), jnp.float32),
            pltpu.VMEM((tail,), jnp.int32),
            pltpu.VMEM((tail, MSG_W), jnp.float32),
            pltpu.SemaphoreType.DMA,
            pltpu.SemaphoreType.DMA,
            pltpu.SemaphoreType.DMA,
            pltpu.SemaphoreType.DMA,
        ],
        compiler_params=_SC_PARAMS,
    )
    def k(msg_hbm, dst_hbm, zer_hbm, out_hbm, acc, idx0, msg0, idx1, msg1,
          idxt_v, msgt_v, sl0, sl1, ss0, ss1):
        cid = lax.axis_index("c")
        sid = lax.axis_index("s")

        @pl.when(sid == 0)
        def _():
            pltpu.sync_copy(zer_hbm, acc)

        plsc.subcore_barrier()
        base = cid * half + sid * per_s

        def fire_load(c, idx_v, msg_v, sem):
            off = base + c * CHUNK
            pltpu.async_copy(dst_hbm.at[pl.ds(off, CHUNK)], idx_v, sem)
            pltpu.async_copy(msg_hbm.at[pl.ds(off, CHUNK)], msg_v, sem)

        def wait_load(idx_v, msg_v, sem):
            pltpu.make_async_copy(dst_hbm.at[pl.ds(base, CHUNK)], idx_v,
                                  sem).wait()
            pltpu.make_async_copy(msg_hbm.at[pl.ds(base, CHUNK)], msg_v,
                                  sem).wait()

        def fire_scatter(idx_v, msg_v, sem):
            pltpu.async_copy(msg_v, acc.at[idx_v], sem, add=True)

        def wait_scatter(idx_v, msg_v, sem):
            pltpu.make_async_copy(msg_v, acc.at[idx_v], sem).wait()

        fire_load(0, idx0, msg0, sl0)
        fire_load(1, idx1, msg1, sl1)

        @pl.loop(0, nch - 2, step=2)
        def _(c):
            wait_load(idx0, msg0, sl0)
            fire_scatter(idx0, msg0, ss0)
            wait_load(idx1, msg1, sl1)
            fire_scatter(idx1, msg1, ss1)
            wait_scatter(idx0, msg0, ss0)
            fire_load(c + 2, idx0, msg0, sl0)
            wait_scatter(idx1, msg1, ss1)
            fire_load(c + 3, idx1, msg1, sl1)

        wait_load(idx0, msg0, sl0)
        fire_scatter(idx0, msg0, ss0)
        wait_load(idx1, msg1, sl1)
        fire_scatter(idx1, msg1, ss1)
        off = base + nch * CHUNK
        pltpu.sync_copy(dst_hbm.at[pl.ds(off, tail)], idxt_v)
        pltpu.sync_copy(msg_hbm.at[pl.ds(off, tail)], msgt_v)
        pltpu.sync_copy(msgt_v, acc.at[idxt_v], add=True)
        wait_scatter(idx1, msg1, ss1)
        wait_scatter(idx0, msg0, ss0)

        plsc.subcore_barrier()

        @pl.when(sid == 0)
        def _():
            pltpu.sync_copy(acc, out_hbm.at[cid])

    return k(msg, dst, zer)


_B_NODE = 2000


def _tc_fin_body(p_ref, q_ref, out_ref):
    s = p_ref[0] + p_ref[1] + q_ref[0] + q_ref[1]
    cnt = jnp.clip(s[:, OUT_W:OUT_W + 1], 1.0, None)
    out_ref[...] = s[:, 0:OUT_W] / cnt


def _tc_finalize(part1, part2):
    spec = pl.BlockSpec((2, _B_NODE, MSG_W), lambda i: (0, i, 0))
    return pl.pallas_call(
        _tc_fin_body,
        grid=(N_NODES // _B_NODE,),
        in_specs=[spec, spec],
        out_specs=pl.BlockSpec((_B_NODE, OUT_W), lambda i: (i, 0)),
        out_shape=jax.ShapeDtypeStruct((N_NODES, OUT_W), jnp.float32),
    )(part1, part2)


def kernel(node_attr, edge_index, edge_attr, edge_sh, fc_w1, fc_b1, fc_w2,
           fc_b2):
    src = edge_index[0]
    dst = edge_index[1]
    node_pad = jnp.pad(node_attr, ((0, 0), (0, 128 - NS)))
    wc, wb = _assemble_weights(fc_w2, fc_b2)
    zer = jnp.zeros((N_NODES, MSG_W), jnp.float32)
    eat, sht, w1t = edge_attr.T, edge_sh.T, fc_w1.T
    e1 = 76800
    e2 = N_EDGES - e1
    x1 = _sc_gather(node_pad, src[:e1], e1)
    msg1 = _tc_msg(x1, eat, sht, w1t, fc_b1, wc, wb, e1, 0)
    x2 = _sc_gather(node_pad, src[e1:], e2)
    part1 = _sc_scatter(msg1, dst[:e1], zer, e1)
    msg2 = _tc_msg(x2, eat, sht, w1t, fc_b1, wc, wb, e2, e1 // _B_EDGE)
    part2 = _sc_scatter(msg2, dst[e1:], zer, e2)
    return _tc_finalize(part1, part2)


# speedup vs baseline: 1.5814x; 1.0023x over previous
import functools

import jax
import jax.numpy as jnp
import numpy as np
from jax import lax
from jax.experimental import pallas as pl
from jax.experimental.pallas import tpu as pltpu
from jax.experimental.pallas import tpu_sc as plsc

NS = 16
NV = 4
N_NODES = 10000
N_EDGES = 160000
SH_DIM = 9
EDGE_FEAT = 3 * NS
MSG_W = 128
OUT_W = NS + 3 * NV + 5 * NV
INV = 1.0 / np.sqrt(NS)

_SC_PARAMS = pltpu.CompilerParams(use_tc_tiling_on_sc=False)

NUM_WORKERS = 32
PER_W = N_EDGES // NUM_WORKERS
CHUNK = 128
NCH = PER_W // CHUNK
TAIL = PER_W - NCH * CHUNK

_E1 = np.zeros((NS, NS * NS), np.float32)
_E2 = np.zeros((NS, NS * NS), np.float32)
for _k in range(NS):
    _E1[_k, _k * NS:(_k + 1) * NS] = 1.0
    for _i in range(NS):
        _E2[_i, _k * NS + _i] = 1.0


_ESH = np.zeros((SH_DIM, MSG_W), np.float32)
_ESH[0, 0:NS] = 1.0
for _v in range(NV):
    for _j in range(3):
        _ESH[1 + _j, NS + _v * 3 + _j] = 1.0
    for _j in range(5):
        _ESH[4 + _j, NS + 3 * NV + _v * 5 + _j] = 1.0

def _assemble_weights(fc_w2, fc_b2):
    w0 = fc_w2[:, :NS * NS].reshape(NS * NS, NS)
    w1 = jnp.repeat(fc_w2[:, NS * NS:NS * NS + NS * NV].reshape(NS * NS, NV),
                    3, axis=1)
    w2 = jnp.repeat(fc_w2[:, NS * NS + NS * NV:].reshape(NS * NS, NV),
                    5, axis=1)
    wc = jnp.concatenate([w0, w1, w2], axis=1) * INV
    b0 = fc_b2[:NS * NS].reshape(NS, NS)
    b1 = jnp.repeat(fc_b2[NS * NS:NS * NS + NS * NV].reshape(NS, NV), 3,
                    axis=1)
    b2 = jnp.repeat(fc_b2[NS * NS + NS * NV:].reshape(NS, NV), 5, axis=1)
    wb = jnp.concatenate([b0, b1, b2], axis=1) * INV
    wc = jnp.pad(wc, ((0, 0), (0, MSG_W - OUT_W)))
    wb = jnp.pad(wb, ((0, 0), (0, MSG_W - OUT_W)))
    return wc, wb


def _sc_gather(node_pad, src, n_e):
    mesh = plsc.VectorSubcoreMesh(core_axis_name="c", subcore_axis_name="s")
    per_w = n_e // NUM_WORKERS
    assert per_w % 8 == 0
    nch = per_w // CHUNK
    assert nch % 2 == 0 and nch >= 2
    tail = per_w - nch * CHUNK

    @functools.partial(
        pl.kernel,
        out_type=jax.ShapeDtypeStruct((n_e, 128), jnp.float32),
        mesh=mesh,
        scratch_types=[
            pltpu.VMEM((CHUNK,), jnp.int32),
            pltpu.VMEM((CHUNK, 128), jnp.float32),
            pltpu.VMEM((CHUNK,), jnp.int32),
            pltpu.VMEM((CHUNK, 128), jnp.float32),
            pltpu.VMEM((tail,), jnp.int32),
            pltpu.VMEM((tail, 128), jnp.float32),
            pltpu.SemaphoreType.DMA,
            pltpu.SemaphoreType.DMA,
            pltpu.SemaphoreType.DMA,
            pltpu.SemaphoreType.DMA,
        ],
        compiler_params=_SC_PARAMS,
    )
    def k(node_hbm, src_hbm, out_hbm, idx0, rows0, idx1, rows1, idxt, rowst,
          sg0, sg1, sw0, sw1):
        cid = lax.axis_index("c")
        sid = lax.axis_index("s")
        base = (sid * 2 + cid) * per_w

        def fire(c, idx_v, rows_v, sg):
            off = base + c * CHUNK
            pltpu.sync_copy(src_hbm.at[pl.ds(off, CHUNK)], idx_v)
            pltpu.async_copy(node_hbm.at[idx_v], rows_v, sg)

        def flush(c, idx_v, rows_v, sg, sw):
            pltpu.make_async_copy(node_hbm.at[idx_v], rows_v, sg).wait()
            off = base + c * CHUNK
            pltpu.async_copy(rows_v, out_hbm.at[pl.ds(off, CHUNK)], sw)

        def wait_wb(rows_v, sw):
            pltpu.make_async_copy(rows_v, out_hbm.at[pl.ds(base, CHUNK)],
                                  sw).wait()

        fire(0, idx0, rows0, sg0)
        fire(1, idx1, rows1, sg1)

        @pl.loop(0, nch - 2, step=2)
        def _(c):
            flush(c, idx0, rows0, sg0, sw0)
            flush(c + 1, idx1, rows1, sg1, sw1)
            wait_wb(rows0, sw0)
            fire(c + 2, idx0, rows0, sg0)
            wait_wb(rows1, sw1)
            fire(c + 3, idx1, rows1, sg1)

        flush(nch - 2, idx0, rows0, sg0, sw0)
        flush(nch - 1, idx1, rows1, sg1, sw1)
        off = base + nch * CHUNK
        pltpu.sync_copy(src_hbm.at[pl.ds(off, tail)], idxt)
        pltpu.async_copy(node_hbm.at[idxt], rowst, sg0).wait()
        pltpu.sync_copy(rowst, out_hbm.at[pl.ds(off, tail)])
        wait_wb(rows1, sw1)
        wait_wb(rows0, sw0)

    return k(node_pad, src)


_B_EDGE = 3200


def _tc_msg_body(x_ref, eat_ref, sht_ref, w1t_ref, b1_ref, e1_ref, e2_ref,
                 wc_ref, wb_ref, esh_ref, out_ref):
    f32 = jnp.float32
    dot = functools.partial(jnp.dot, preferred_element_type=f32)
    ht = jnp.maximum(dot(w1t_ref[...], eat_ref[...]) + b1_ref[...], 0.0)
    h = ht.T
    sh = sht_ref[...].T
    x16 = x_ref[:, 0:NS]
    he = dot(h, e1_ref[...])
    xe = dot(x16, e2_ref[...])
    pre = dot(xe * he, wc_ref[...]) + dot(x16, wb_ref[...])
    s = dot(sh, esh_ref[...])
    lane = lax.broadcasted_iota(jnp.int32, out_ref.shape, 1)
    out_ref[...] = pre * s + jnp.where(lane == OUT_W, 1.0, 0.0)


def _tc_msg(x128, eat, sht, w1t, fc_b1, wc, wb, n_e, blk_off):
    grid = (n_e // _B_EDGE,)
    full = lambda i: (0, 0)
    return pl.pallas_call(
        _tc_msg_body,
        grid=grid,
        in_specs=[
            pl.BlockSpec((_B_EDGE, 128), lambda i: (i, 0)),
            pl.BlockSpec((EDGE_FEAT, _B_EDGE), lambda i: (0, i + blk_off)),
            pl.BlockSpec((SH_DIM, _B_EDGE), lambda i: (0, i + blk_off)),
            pl.BlockSpec((NS, EDGE_FEAT), full),
            pl.BlockSpec((NS, 1), full),
            pl.BlockSpec((NS, NS * NS), full),
            pl.BlockSpec((NS, NS * NS), full),
            pl.BlockSpec((NS * NS, MSG_W), full),
            pl.BlockSpec((NS, MSG_W), full),
            pl.BlockSpec((SH_DIM, MSG_W), full),
        ],
        out_specs=pl.BlockSpec((_B_EDGE, MSG_W), lambda i: (i, 0)),
        out_shape=jax.ShapeDtypeStruct((n_e, MSG_W), jnp.float32),
    )(x128, eat, sht, w1t, fc_b1.reshape(NS, 1), jnp.asarray(_E1),
      jnp.asarray(_E2), wc, wb, jnp.asarray(_ESH))


def _sc_scatter(msg, dst, zer, n_e):
    mesh = plsc.VectorSubcoreMesh(core_axis_name="c", subcore_axis_name="s")
    half = n_e // 2
    per_s = half // 16
    assert per_s % 8 == 0
    nch = per_s // CHUNK
    assert nch % 2 == 0 and nch >= 2
    tail = per_s - nch * CHUNK

    @functools.partial(
        pl.kernel,
        out_type=jax.ShapeDtypeStruct((2, N_NODES, MSG_W), jnp.float32),
        mesh=mesh,
        scratch_types=[
            pltpu.VMEM_SHARED((N_NODES, MSG_W), jnp.float32),
            pltpu.VMEM((CHUNK,), jnp.int32),
            pltpu.VMEM((CHUNK, MSG_W), jnp.float32),
            pltpu.VMEM((CHUNK,), jnp.int32),
            pltpu.VMEM((CHUNK, MSG_W), jnp.float32),
            pltpu.VMEM((tail,), jnp.int32),
            pltpu.VMEM((tail, MSG_W), jnp.float32),
            pltpu.SemaphoreType.DMA,
            pltpu.SemaphoreType.DMA,
            pltpu.SemaphoreType.DMA,
            pltpu.SemaphoreType.DMA,
        ],
        compiler_params=_SC_PARAMS,
    )
    def k(msg_hbm, dst_hbm, zer_hbm, out_hbm, acc, idx0, msg0, idx1, msg1,
          idxt_v, msgt_v, sl0, sl1, ss0, ss1):
        cid = lax.axis_index("c")
        sid = lax.axis_index("s")

        @pl.when(sid == 0)
        def _():
            pltpu.sync_copy(zer_hbm, acc)

        plsc.subcore_barrier()
        base = cid * half + sid * per_s

        def fire_load(c, idx_v, msg_v, sem):
            off = base + c * CHUNK
            pltpu.async_copy(dst_hbm.at[pl.ds(off, CHUNK)], idx_v, sem)
            pltpu.async_copy(msg_hbm.at[pl.ds(off, CHUNK)], msg_v, sem)

        def wait_load(idx_v, msg_v, sem):
            pltpu.make_async_copy(dst_hbm.at[pl.ds(base, CHUNK)], idx_v,
                                  sem).wait()
            pltpu.make_async_copy(msg_hbm.at[pl.ds(base, CHUNK)], msg_v,
                                  sem).wait()

        def fire_scatter(idx_v, msg_v, sem):
            pltpu.async_copy(msg_v, acc.at[idx_v], sem, add=True)

        def wait_scatter(idx_v, msg_v, sem):
            pltpu.make_async_copy(msg_v, acc.at[idx_v], sem).wait()

        fire_load(0, idx0, msg0, sl0)
        fire_load(1, idx1, msg1, sl1)

        @pl.loop(0, nch - 2, step=2)
        def _(c):
            wait_load(idx0, msg0, sl0)
            fire_scatter(idx0, msg0, ss0)
            wait_load(idx1, msg1, sl1)
            fire_scatter(idx1, msg1, ss1)
            wait_scatter(idx0, msg0, ss0)
            fire_load(c + 2, idx0, msg0, sl0)
            wait_scatter(idx1, msg1, ss1)
            fire_load(c + 3, idx1, msg1, sl1)

        wait_load(idx0, msg0, sl0)
        fire_scatter(idx0, msg0, ss0)
        wait_load(idx1, msg1, sl1)
        fire_scatter(idx1, msg1, ss1)
        off = base + nch * CHUNK
        pltpu.sync_copy(dst_hbm.at[pl.ds(off, tail)], idxt_v)
        pltpu.sync_copy(msg_hbm.at[pl.ds(off, tail)], msgt_v)
        pltpu.sync_copy(msgt_v, acc.at[idxt_v], add=True)
        wait_scatter(idx1, msg1, ss1)
        wait_scatter(idx0, msg0, ss0)

        plsc.subcore_barrier()

        @pl.when(sid == 0)
        def _():
            pltpu.sync_copy(acc, out_hbm.at[cid])

    return k(msg, dst, zer)


_B_NODE = 2000


def _tc_fin_body(p_ref, q_ref, out_ref):
    s = p_ref[0] + p_ref[1] + q_ref[0] + q_ref[1]
    cnt = jnp.clip(s[:, OUT_W:OUT_W + 1], 1.0, None)
    out_ref[...] = s[:, 0:OUT_W] / cnt


def _tc_finalize(part1, part2):
    spec = pl.BlockSpec((2, _B_NODE, MSG_W), lambda i: (0, i, 0))
    return pl.pallas_call(
        _tc_fin_body,
        grid=(N_NODES // _B_NODE,),
        in_specs=[spec, spec],
        out_specs=pl.BlockSpec((_B_NODE, OUT_W), lambda i: (i, 0)),
        out_shape=jax.ShapeDtypeStruct((N_NODES, OUT_W), jnp.float32),
    )(part1, part2)


def kernel(node_attr, edge_index, edge_attr, edge_sh, fc_w1, fc_b1, fc_w2,
           fc_b2):
    src = edge_index[0]
    dst = edge_index[1]
    node_pad = jnp.pad(node_attr, ((0, 0), (0, 128 - NS)))
    wc, wb = _assemble_weights(fc_w2, fc_b2)
    zer = jnp.zeros((N_NODES, MSG_W), jnp.float32)
    eat, sht, w1t = edge_attr.T, edge_sh.T, fc_w1.T
    e1 = 76800
    e2 = N_EDGES - e1
    x1 = _sc_gather(node_pad, src[:e1], e1)
    msg1 = _tc_msg(x1, eat, sht, w1t, fc_b1, wc, wb, e1, 0)
    x2 = _sc_gather(node_pad, src[e1:], e2)
    part1 = _sc_scatter(msg1, dst[:e1], zer, e1)
    msg2 = _tc_msg(x2, eat, sht, w1t, fc_b1, wc, wb, e2, e1 // _B_EDGE)
    part2 = _sc_scatter(msg2, dst[e1:], zer, e2)
    return _tc_finalize(part1, part2)
